# baseline (speedup 1.0000x reference)
import functools

import jax
import jax.numpy as jnp
from jax.experimental import pallas as pl

_ALPHA = 0.25
_B = 4096


def _loss_kernel(num_blocks, num_classes,
                 x_ref, ctc_ref, mskc_ref, pb_ref, bt_ref, ctr_ref,
                 pi_ref, ti_ref,
                 cls_ref, reg_ref, iou_ref, fg_ref):
    g = pl.program_id(0)

    x = x_ref[...]
    ct = ctc_ref[...]
    mval = mskc_ref[...]
    valid = jnp.where(ct >= 0, 1.0, 0.0) * (1.0 - mval)

    lane = jax.lax.broadcasted_iota(jnp.int32, x.shape, 1)
    tau = lane == ct
    a = jnp.exp(-jnp.abs(x))
    l1p = jnp.log1p(a)
    ce0 = jnp.maximum(x, 0.0) + l1p
    tx = jnp.where(tau, x, 0.0)
    ce = ce0 - tx
    arg = (ce0 - x) + tx
    fsq = jnp.exp(-2.0 * arg)
    at = jnp.where(tau, _ALPHA, 1.0 - _ALPHA)
    cls_part = jnp.sum((at * ce * fsq) * valid)

    ctr = ctr_ref[0]
    fg = jnp.where((ctr >= 0) & (ctr != num_classes), 1.0, 0.0)
    fg_part = jnp.sum(fg)

    pb = pb_ref[...]
    bt = bt_ref[...]
    px0, py0, px1, py1 = pb[0:1], pb[1:2], pb[2:3], pb[3:4]
    tx0, ty0, tx1, ty1 = bt[0:1], bt[1:2], bt[2:3], bt[3:4]
    area1 = (px1 - px0) * (py1 - py0)
    area2 = (tx1 - tx0) * (ty1 - ty0)
    iw = jnp.clip(jnp.minimum(px1, tx1) - jnp.maximum(px0, tx0), 0.0, None)
    ih = jnp.clip(jnp.minimum(py1, ty1) - jnp.maximum(py0, ty0), 0.0, None)
    inter = iw * ih
    union = area1 + area2 - inter
    ew = jnp.maximum(px1, tx1) - jnp.minimum(px0, tx0)
    eh = jnp.maximum(py1, ty1) - jnp.minimum(py0, ty0)
    area_e = jnp.clip(ew, 0.0, None) * jnp.clip(eh, 0.0, None)
    giou = inter / union - (area_e - union) / area_e
    reg_part = jnp.sum((1.0 - giou) * fg)

    pi = pi_ref[0]
    ti = ti_ref[0]
    bce = jnp.maximum(pi, 0.0) - pi * ti + jnp.log1p(jnp.exp(-jnp.abs(pi)))
    iou_part = jnp.sum(bce * fg)

    one = jnp.ones((1, 1), jnp.float32)

    @pl.when(g == 0)
    def _():
        z = jnp.zeros((1, 1), jnp.float32)
        cls_ref[...] = z
        reg_ref[...] = z
        iou_ref[...] = z
        fg_ref[...] = z

    cls_t = cls_ref[...] + cls_part * one
    reg_t = reg_ref[...] + reg_part * one
    iou_t = iou_ref[...] + iou_part * one
    fg_t = fg_ref[...] + fg_part * one
    cls_ref[...] = cls_t
    reg_ref[...] = reg_t
    iou_ref[...] = iou_t
    fg_ref[...] = fg_t

    @pl.when(g == num_blocks - 1)
    def _():
        nf = jnp.maximum(fg_t, 1.0)
        cls_ref[...] = cls_t / nf
        reg_ref[...] = reg_t / nf
        iou_ref[...] = iou_t / nf


def kernel(pred_cls, pred_box, pred_iou, cls_targets, box_targets,
           iou_targets, mask):
    n, c = pred_cls.shape
    b = _B
    nb = n // b
    ct = cls_targets.astype(jnp.int32)
    ct_col = ct.reshape(n, 1)
    mask_col = mask.astype(jnp.float32).reshape(n, 1)
    pb_t = pred_box.T
    bt_t = box_targets.T
    ct_row = ct.reshape(nb, 1, b)
    pi_row = pred_iou.reshape(nb, 1, b)
    ti_row = iou_targets.reshape(nb, 1, b)

    out = pl.pallas_call(
        functools.partial(_loss_kernel, nb, c),
        grid=(nb,),
        in_specs=[
            pl.BlockSpec((b, c), lambda g: (g, 0)),
            pl.BlockSpec((b, 1), lambda g: (g, 0)),
            pl.BlockSpec((b, 1), lambda g: (g, 0)),
            pl.BlockSpec((4, b), lambda g: (0, g)),
            pl.BlockSpec((4, b), lambda g: (0, g)),
            pl.BlockSpec((1, 1, b), lambda g: (g, 0, 0)),
            pl.BlockSpec((1, 1, b), lambda g: (g, 0, 0)),
            pl.BlockSpec((1, 1, b), lambda g: (g, 0, 0)),
        ],
        out_specs=[
            pl.BlockSpec((1, 1), lambda g: (0, 0)),
            pl.BlockSpec((1, 1), lambda g: (0, 0)),
            pl.BlockSpec((1, 1), lambda g: (0, 0)),
            pl.BlockSpec((1, 1), lambda g: (0, 0)),
        ],
        out_shape=[jax.ShapeDtypeStruct((1, 1), jnp.float32)] * 4,
    )(pred_cls, ct_col, mask_col, pb_t, bt_t, ct_row, pi_row, ti_row)
    cls_s, reg_s, iou_s, _ = out
    return (cls_s[0, 0], reg_s[0, 0], iou_s[0, 0])


# speedup vs baseline: 1.3342x; 1.3342x over previous
import functools

import jax
import jax.numpy as jnp
from jax.experimental import pallas as pl
from jax.experimental.pallas import tpu as pltpu

_ALPHA = 0.25
_B = 4096


def _loss_kernel(num_blocks, num_classes,
                 x_ref, ctm_ref, pb_ref, bt_ref, pi_ref, ti_ref,
                 cls_ref, reg_ref, iou_ref, acc_ref):
    g = pl.program_id(0)

    ct = ctm_ref[0, 0:1, :]
    mval = ctm_ref[0, 1:2, :]
    fg = jnp.where((ct >= 0.0) & (ct != float(num_classes)), 1.0, 0.0)
    valid = jnp.where(ct >= 0.0, 1.0, 0.0) * (1.0 - mval)
    fg_part = jnp.sum(fg)

    ctv = jnp.concatenate([ct, valid], axis=0)
    ctv_col = ctv.T
    ct_col = ctv_col[:, 0:1]
    valid_col = ctv_col[:, 1:2]

    x = x_ref[...]
    lane = jax.lax.broadcasted_iota(jnp.int32, x.shape, 1).astype(jnp.float32)
    tau = lane == ct_col
    a = jnp.exp(-jnp.abs(x))
    l1p = jnp.log1p(a)
    ce0 = jnp.maximum(x, 0.0) + l1p
    tx = jnp.where(tau, x, 0.0)
    ce = ce0 - tx
    arg = (ce0 - x) + tx
    fsq = jnp.exp(-2.0 * arg)
    at = jnp.where(tau, _ALPHA, 1.0 - _ALPHA)
    cls_part = jnp.sum((at * ce * fsq) * valid_col)

    pb = pb_ref[...]
    bt = bt_ref[...]
    px0, py0, px1, py1 = pb[0:1], pb[1:2], pb[2:3], pb[3:4]
    tx0, ty0, tx1, ty1 = bt[0:1], bt[1:2], bt[2:3], bt[3:4]
    area1 = (px1 - px0) * (py1 - py0)
    area2 = (tx1 - tx0) * (ty1 - ty0)
    iw = jnp.clip(jnp.minimum(px1, tx1) - jnp.maximum(px0, tx0), 0.0, None)
    ih = jnp.clip(jnp.minimum(py1, ty1) - jnp.maximum(py0, ty0), 0.0, None)
    inter = iw * ih
    union = area1 + area2 - inter
    ew = jnp.maximum(px1, tx1) - jnp.minimum(px0, tx0)
    eh = jnp.maximum(py1, ty1) - jnp.minimum(py0, ty0)
    area_e = jnp.clip(ew, 0.0, None) * jnp.clip(eh, 0.0, None)
    giou = inter / union - (area_e - union) / area_e
    reg_part = jnp.sum((1.0 - giou) * fg)

    pi = pi_ref[0]
    ti = ti_ref[0]
    bce = jnp.maximum(pi, 0.0) - pi * ti + jnp.log1p(jnp.exp(-jnp.abs(pi)))
    iou_part = jnp.sum(bce * fg)

    part = jnp.concatenate(
        [jnp.full((1, 1), v, jnp.float32)
         for v in (cls_part, reg_part, iou_part, fg_part)], axis=1)

    @pl.when(g == 0)
    def _():
        acc_ref[...] = jnp.zeros((1, 4), jnp.float32)

    tot = acc_ref[...] + part
    acc_ref[...] = tot

    @pl.when(g == num_blocks - 1)
    def _():
        nf = jnp.maximum(tot[0:1, 3:4], 1.0)
        cls_ref[...] = tot[0:1, 0:1] / nf
        reg_ref[...] = tot[0:1, 1:2] / nf
        iou_ref[...] = tot[0:1, 2:3] / nf


def kernel(pred_cls, pred_box, pred_iou, cls_targets, box_targets,
           iou_targets, mask):
    n, c = pred_cls.shape
    b = _B
    nb = n // b
    ctm = jnp.stack([cls_targets.astype(jnp.float32),
                     mask.astype(jnp.float32)], axis=0)
    ctm = ctm.reshape(2, nb, b).transpose(1, 0, 2)
    pb_t = pred_box.T
    bt_t = box_targets.T
    pi_row = pred_iou.reshape(nb, 1, b)
    ti_row = iou_targets.reshape(nb, 1, b)

    out = pl.pallas_call(
        functools.partial(_loss_kernel, nb, c),
        grid=(nb,),
        in_specs=[
            pl.BlockSpec((b, c), lambda g: (g, 0)),
            pl.BlockSpec((1, 2, b), lambda g: (g, 0, 0)),
            pl.BlockSpec((4, b), lambda g: (0, g)),
            pl.BlockSpec((4, b), lambda g: (0, g)),
            pl.BlockSpec((1, 1, b), lambda g: (g, 0, 0)),
            pl.BlockSpec((1, 1, b), lambda g: (g, 0, 0)),
        ],
        out_specs=[
            pl.BlockSpec((1, 1), lambda g: (0, 0)),
            pl.BlockSpec((1, 1), lambda g: (0, 0)),
            pl.BlockSpec((1, 1), lambda g: (0, 0)),
        ],
        out_shape=[jax.ShapeDtypeStruct((1, 1), jnp.float32)] * 3,
        scratch_shapes=[pltpu.VMEM((1, 4), jnp.float32)],
    )(pred_cls, ctm, pb_t, bt_t, pi_row, ti_row)
    cls_s, reg_s, iou_s = out
    return (cls_s[0, 0], reg_s[0, 0], iou_s[0, 0])
